# baseline (speedup 1.0000x reference)
import functools

import jax
import jax.numpy as jnp
from jax import lax
from jax.experimental import pallas as pl
from jax.experimental.pallas import tpu as pltpu
from jax.experimental.pallas import tpu_sc as plsc

N_TOTAL = 60000
N_NODES = 50000
E = 960000
D_IN = 128
H = 64
C = 40

HALF = H // 2
NC, NS = 2, 16
CH = 96
SG = 4
CHUNKS = 10240
E_PAD = CHUNKS * CH
CPT = CHUNKS // NS
STAGES = CPT // SG
ROWS_A = 3752
ROWS_LAST = N_TOTAL - 15 * ROWS_A


def _mm1(x, W1, b1):
    BM = 2000
    nb = N_TOTAL // BM

    def body(x_ref, w_ref, b_ref, o_ref):
        o_ref[...] = jnp.dot(
            x_ref[...], w_ref[0], preferred_element_type=jnp.float32,
            precision=lax.Precision.HIGHEST) + b_ref[0]

    w_split = W1.reshape(D_IN, NC, HALF).transpose(1, 0, 2)
    return pl.pallas_call(
        body,
        grid=(NC, nb),
        in_specs=[
            pl.BlockSpec((BM, D_IN), lambda c, i: (i, 0)),
            pl.BlockSpec((1, D_IN, HALF), lambda c, i: (c, 0, 0)),
            pl.BlockSpec((1, 1, HALF), lambda c, i: (c, 0, 0)),
        ],
        out_specs=pl.BlockSpec((BM, HALF), lambda c, i: (c * nb + i, 0)),
        out_shape=jax.ShapeDtypeStruct((NC * N_TOTAL, HALF), jnp.float32),
    )(x, w_split, b1.reshape(NC, 1, HALF))


def _propagate(emb0, pkp, zeros_hbm):
    mesh = plsc.VectorSubcoreMesh(core_axis_name="c", subcore_axis_name="s")
    out_t = [jax.ShapeDtypeStruct((NC, N_TOTAL, HALF), jnp.float32)] * 3

    @functools.partial(
        pl.kernel, mesh=mesh, out_type=out_t,
        compiler_params=pltpu.CompilerParams(
            use_tc_tiling_on_sc=False, needs_layout_passes=False),
        scratch_types=[
            pltpu.VMEM_SHARED((N_TOTAL, HALF), jnp.float32),
            pltpu.VMEM((SG, 3, CH), jnp.int32),
            pltpu.VMEM((SG, 3, CH), jnp.int32),
            pltpu.VMEM((CH, HALF), jnp.float32),
            pltpu.VMEM((CH, HALF), jnp.float32),
            pltpu.SemaphoreType.DMA((2,)),
            pltpu.SemaphoreType.DMA((2,)),
            pltpu.SemaphoreType.DMA((2,)),
        ],
    )
    def prop(e0, pkr, zr, o1, o2, o3,
             acc, pk0, pk1, rows0, rows1, isem, gsem, ssem):
        pks = (pk0, pk1)
        rows = (rows0, rows1)
        c = lax.axis_index("c")
        t = lax.axis_index("s")
        lane_ids = [jnp.full((16, 1), l, dtype=jnp.int32) for l in range(16)]
        bcast_dnums = lax.GatherDimensionNumbers(
            offset_dims=(), collapsed_slice_dims=(0,), start_index_map=(0,))

        def scale(rj, pk, j):
            @pl.loop(0, CH, step=16)
            def _(i0):
                wvec = plsc.bitcast(pk[j, 2, pl.ds(i0, 16)], jnp.float32)
                for l in range(16):
                    i = i0 + l
                    wb = lax.gather(
                        wvec, lane_ids[l], bcast_dnums, (1,),
                        mode=lax.GatherScatterMode.PROMISE_IN_BOUNDS)
                    for kk in range(0, HALF, 16):
                        rj[i, pl.ds(kk, 16)] = rj[i, pl.ds(kk, 16)] * wb

        def wait_rowsz(sem):
            pltpu.make_async_copy(zr.at[pl.ds(0, CH)], rows0, sem).wait()

        def layer(emb_hbm, out_hbm):
            emb_c = emb_hbm.at[c]
            out_c = out_hbm.at[c]

            @pl.when(t < NS - 1)
            def _():
                pltpu.sync_copy(zr, acc.at[pl.ds(t * ROWS_A, ROWS_A)])

            @pl.when(t == NS - 1)
            def _():
                pltpu.sync_copy(zr.at[pl.ds(0, ROWS_LAST)],
                                acc.at[pl.ds(15 * ROWS_A, ROWS_LAST)])
            plsc.subcore_barrier()

            base = t * CPT
            pltpu.async_copy(pkr.at[pl.ds(base, SG)], pk0, isem.at[0]).wait()
            pltpu.async_copy(pkr.at[pl.ds(base + SG, SG)], pk1, isem.at[1])
            pltpu.async_copy(emb_c.at[pk0.at[0, 0]], rows0, gsem.at[0])

            @pl.loop(0, STAGES // 2)
            def _(u):
                for half in range(2):
                    s = 2 * u + half
                    pk, pko = pks[half], pks[1 - half]
                    srow = base + s * SG
                    for j in range(SG):
                        b = j % 2
                        if j == 0:
                            @pl.when(s >= 1)
                            def _():
                                wait_rowsz(ssem.at[1])

                            @pl.when((s >= 1) & (s + 1 < STAGES))
                            def _():
                                pltpu.async_copy(
                                    pkr.at[pl.ds(srow + SG, SG)], pko,
                                    isem.at[1 - half])
                            pltpu.async_copy(
                                emb_c.at[pk.at[1, 0]], rows1, gsem.at[1])
                        elif j < SG - 1:
                            wait_rowsz(ssem.at[1 - b])
                            pltpu.async_copy(
                                emb_c.at[pk.at[j + 1, 0]], rows[1 - b],
                                gsem.at[1 - b])
                        else:
                            wait_rowsz(ssem.at[0])

                            @pl.when(s + 1 < STAGES)
                            def _():
                                pltpu.make_async_copy(
                                    pkr.at[pl.ds(base, SG)], pko,
                                    isem.at[1 - half]).wait()
                                pltpu.async_copy(
                                    emb_c.at[pko.at[0, 0]], rows0,
                                    gsem.at[0])
                        wait_rowsz(gsem.at[b])
                        scale(rows[b], pk, j)
                        pltpu.async_copy(
                            rows[b], acc.at[pk.at[j, 1]], ssem.at[b],
                            add=True)
            wait_rowsz(ssem.at[1])
            plsc.subcore_barrier()

            @pl.when(t < NS - 1)
            def _():
                pltpu.sync_copy(
                    acc.at[pl.ds(t * ROWS_A, ROWS_A)],
                    out_c.at[pl.ds(t * ROWS_A, ROWS_A)])

            @pl.when(t == NS - 1)
            def _():
                pltpu.sync_copy(
                    acc.at[pl.ds(15 * ROWS_A, ROWS_LAST)],
                    out_c.at[pl.ds(15 * ROWS_A, ROWS_LAST)])
            plsc.subcore_barrier()

        layer(e0, o1)
        layer(o1, o2)
        layer(o2, o3)

    return prop(emb0, pkp, zeros_hbm)


def _head(e0, e1, e2, e3, W2, b2):
    BN = 1000
    nb = N_NODES // BN

    def body(a_ref, b_ref, c_ref, d_ref, w_ref, bias_ref, o_ref):
        m = (a_ref[...] + b_ref[...] + c_ref[...] + d_ref[...]) * 0.25
        cat = jnp.concatenate([m[0], m[1]], axis=1)
        z = jnp.dot(cat, w_ref[...], preferred_element_type=jnp.float32,
                    precision=lax.Precision.HIGHEST) + bias_ref[...]
        zmax = jnp.max(z, axis=1, keepdims=True)
        lse = jnp.log(jnp.sum(jnp.exp(z - zmax), axis=1, keepdims=True)) + zmax
        o_ref[...] = z - lse

    espec = pl.BlockSpec((NC, BN, HALF), lambda i: (0, i, 0))
    args = [e0.reshape(NC, N_TOTAL, HALF), e1, e2, e3]
    return pl.pallas_call(
        body,
        grid=(nb,),
        in_specs=[espec] * 4 + [
            pl.BlockSpec((H, C), lambda i: (0, 0)),
            pl.BlockSpec((1, C), lambda i: (0, 0)),
        ],
        out_specs=pl.BlockSpec((BN, C), lambda i: (i, 0)),
        out_shape=jax.ShapeDtypeStruct((N_NODES, C), jnp.float32),
    )(*args, W2, b2.reshape(1, C))


def kernel(x, edge_index, edge_weight, W1, b1, W2, b2):
    src = edge_index[0].astype(jnp.int32)
    dst = edge_index[1].astype(jnp.int32)
    wbits = lax.bitcast_convert_type(edge_weight.astype(jnp.float32),
                                     jnp.int32)
    pad = E_PAD - E
    zpad = jnp.zeros((pad,), jnp.int32)
    pk = jnp.stack([
        jnp.concatenate([src, zpad]).reshape(CHUNKS, CH),
        jnp.concatenate([dst, zpad]).reshape(CHUNKS, CH),
        jnp.concatenate([wbits, zpad]).reshape(CHUNKS, CH),
    ], axis=1)
    emb0 = _mm1(x, W1, b1)
    zeros_hbm = jnp.zeros((ROWS_A, HALF), jnp.float32)
    e1, e2, e3 = _propagate(emb0.reshape(NC, N_TOTAL, HALF), pk, zeros_hbm)
    return _head(emb0, e1, e2, e3, W2, b2)
